# baseline (speedup 1.0000x reference)
import functools

import jax
import jax.numpy as jnp
from jax import lax
from jax.experimental import pallas as pl
from jax.experimental.pallas import tpu as pltpu
from jax.experimental.pallas import tpu_sc as plsc

B = 4096
K = 64
D = 64
GAMMA = 12.0

NC = 2
NS = 16
L = 16
NW = NC * NS
NB = B // NW
GU = 4
ROWS_PER_UNIT = GU * K
NU = NB // GU
NBUF = 3


def _logsig(x):
    ax = jnp.abs(x)
    z = jnp.exp(-ax)
    w = z / (2.0 + z)
    u = w * w
    p = 1.0 + u * (1.0 / 3.0 + u * (0.2 + u * (1.0 / 7.0
        + u * (1.0 / 9.0 + u * (1.0 / 11.0)))))
    return jnp.minimum(x, 0.0) - 2.0 * w * p


def _pack16(vals, iota):
    acc = jnp.full((L,), vals[0], jnp.float32)
    for kk in range(1, L):
        acc = jnp.where(iota == kk, vals[kk], acc)
    return acc


def _row_l1(ref, row, hr):
    acc = jnp.abs(hr[0] - ref[row, pl.ds(0, L)])
    for c in range(1, D // L):
        acc = acc + jnp.abs(hr[c] - ref[row, pl.ds(c * L, L)])
    return jnp.sum(acc)


def _body(hidx_h, ridx_h, tidx_h, nidx_h, ent_h, rel_h,
          pos_h, neg_h,
          hidx_v, ridx_v, tidx_v, nidx_v,
          hrows, rrows, trows, negb,
          partials, posb, negbuf_out,
          sem_h, sem_r, sem_t, sem0, sem1, sem2, sem3):
    sems = (sem0, sem1, sem2, sem3)
    cid = lax.axis_index("c")
    sid = lax.axis_index("s")
    wid = sid * NC + cid
    base = wid * NB

    pltpu.sync_copy(hidx_h.at[pl.ds(base, NB)], hidx_v)
    pltpu.sync_copy(ridx_h.at[pl.ds(base, NB)], ridx_v)
    pltpu.sync_copy(tidx_h.at[pl.ds(base, NB)], tidx_v)
    pltpu.sync_copy(nidx_h.at[pl.ds(wid * NU, NU)], nidx_v)

    cp_h = pltpu.async_copy(ent_h.at[hidx_v], hrows, sem_h)
    cp_r = pltpu.async_copy(rel_h.at[ridx_v], rrows, sem_r)
    cp_t = pltpu.async_copy(ent_h.at[tidx_v], trows, sem_t)
    for u in range(NBUF - 1):
        pltpu.async_copy(ent_h.at[nidx_v.at[u]], negb.at[u], sems[u])

    cp_h.wait()
    cp_r.wait()
    iota = lax.iota(jnp.int32, L)

    def _hr_body(b, carry):
        for c in range(D // L):
            rrows[b, pl.ds(c * L, L)] = (
                rrows[b, pl.ds(c * L, L)] + hrows[b, pl.ds(c * L, L)])
        return carry
    lax.fori_loop(0, NB, _hr_body, 0)

    cp_t.wait()

    def _pos_body(g, carry):
        vals = []
        for kk in range(L):
            b = g * L + kk
            hr = [rrows[b, pl.ds(c * L, L)] for c in range(D // L)]
            vals.append(_row_l1(trows, b, hr))
        acc = _pack16(vals, iota)
        posb[pl.ds(g * L, L)] = _logsig(GAMMA - acc)
        return carry
    lax.fori_loop(0, NB // L, _pos_body, 0)

    def _unit(u, nb):
        pltpu.make_async_copy(ent_h.at[nidx_v.at[0]], negb.at[nb], sems[nb]).wait()
        buf = negb.at[nb]
        for j in range(GU):
            bl = u * GU + j
            hr = [rrows[bl, pl.ds(c * L, L)] for c in range(D // L)]

            def _gblk(g, lsacc):
                row0 = j * K + g * L
                vals = []
                for kk in range(L):
                    vals.append(_row_l1(buf, row0 + kk, hr))
                acc = _pack16(vals, iota)
                return lsacc + _logsig(acc - GAMMA)

            lsacc = lax.fori_loop(0, K // L, _gblk, jnp.zeros((L,), jnp.float32))
            partials[bl, :] = lsacc

        nxt = u + NBUF - 1
        slot = (nb + NBUF - 1) % NBUF

        if isinstance(nxt, int):
            if nxt < NU:
                pltpu.async_copy(ent_h.at[nidx_v.at[nxt]], negb.at[slot],
                                 sems[slot])
        else:
            @pl.when(nxt < NU)
            def _():
                pltpu.async_copy(ent_h.at[nidx_v.at[nxt]], negb.at[slot],
                                 sems[slot])

    def _ring_body(u0, carry):
        for nb in range(NBUF):
            _unit(u0 * NBUF + nb, nb)
        return carry
    lax.fori_loop(0, NU // NBUF, _ring_body, 0)

    for r in range(NU - (NU // NBUF) * NBUF):
        _unit((NU // NBUF) * NBUF + r, r)

    def _fin_body(g, carry):
        vals = []
        for kk in range(L):
            vals.append(jnp.sum(partials[g * L + kk, :]))
        negbuf_out[pl.ds(g * L, L)] = _pack16(vals, iota) * jnp.float32(1.0 / K)
        return carry
    lax.fori_loop(0, NB // L, _fin_body, 0)

    pltpu.sync_copy(posb, pos_h.at[pl.ds(base, NB)])
    pltpu.sync_copy(negbuf_out, neg_h.at[pl.ds(base, NB)])


_sc_score = functools.partial(
    pl.kernel,
    out_type=(jax.ShapeDtypeStruct((B,), jnp.float32),
              jax.ShapeDtypeStruct((B,), jnp.float32)),
    mesh=plsc.VectorSubcoreMesh(core_axis_name="c", subcore_axis_name="s",
                                num_cores=NC, num_subcores=NS),
    compiler_params=pltpu.CompilerParams(needs_layout_passes=False,
                                         use_tc_tiling_on_sc=False),
    scratch_types=[
        pltpu.VMEM((NB,), jnp.int32),
        pltpu.VMEM((NB,), jnp.int32),
        pltpu.VMEM((NB,), jnp.int32),
        pltpu.VMEM((NU, ROWS_PER_UNIT), jnp.int32),
        pltpu.VMEM((NB, D), jnp.float32),
        pltpu.VMEM((NB, D), jnp.float32),
        pltpu.VMEM((NB, D), jnp.float32),
        pltpu.VMEM((NBUF, ROWS_PER_UNIT, D), jnp.float32),
        pltpu.VMEM((NB, L), jnp.float32),
        pltpu.VMEM((NB,), jnp.float32),
        pltpu.VMEM((NB,), jnp.float32),
        pltpu.SemaphoreType.DMA,
        pltpu.SemaphoreType.DMA,
        pltpu.SemaphoreType.DMA,
        pltpu.SemaphoreType.DMA,
        pltpu.SemaphoreType.DMA,
        pltpu.SemaphoreType.DMA,
        pltpu.SemaphoreType.DMA,
    ],
)(_body)


TR_LOG = 15
TR_C = 1 << TR_LOG
TR_H = TR_C // 2


def _tr_body(in_ref, out_ref):
    t = in_ref[...]
    out_ref[...] = jnp.concatenate([t[:, :TR_H], t[:, TR_H:]], axis=0).T


def _transpose_pack(entT):
    n = entT.shape[1]
    nblk = pl.cdiv(n, TR_C)
    return pl.pallas_call(
        _tr_body,
        grid=(nblk,),
        in_specs=[pl.BlockSpec((D, TR_C), lambda i: (0, i))],
        out_specs=pl.BlockSpec((TR_H, 128), lambda i: (i, 0)),
        out_shape=jax.ShapeDtypeStruct((nblk * TR_H, 128), jnp.float32),
        compiler_params=pltpu.CompilerParams(
            dimension_semantics=("parallel",)),
    )(entT)


def _remap(e):
    e = e.astype(jnp.int32)
    return (((e >> TR_LOG) << TR_LOG) + ((e & (TR_H - 1)) << 1)
            + ((e >> (TR_LOG - 1)) & 1))


def kernel(head_idx, rel_idx, tail_idx, neg_idx, entity_emb, relation_emb):
    ent128 = _transpose_pack(entity_emb.T)
    ent2 = ent128.reshape(2 * ent128.shape[0], D)
    neg_idx = _remap(neg_idx).reshape(B * K // ROWS_PER_UNIT, ROWS_PER_UNIT)
    pos, neg = _sc_score(_remap(head_idx), rel_idx.astype(jnp.int32),
                         _remap(tail_idx), neg_idx, ent2, relation_emb)
    return jnp.stack([pos, neg], axis=1)


# speedup vs baseline: 1.0271x; 1.0271x over previous
import functools

import jax
import jax.numpy as jnp
from jax import lax
from jax.experimental import pallas as pl
from jax.experimental.pallas import tpu as pltpu
from jax.experimental.pallas import tpu_sc as plsc

B = 4096
K = 64
D = 64
GAMMA = 12.0

NC = 2
NS = 16
L = 16
NW = NC * NS
NB = B // NW
GU = 2
ROWS_PER_UNIT = GU * K
NU = NB // GU
NBUF = 5


def _logsig(x):
    ax = jnp.abs(x)
    z = jnp.exp(-ax)
    w = z / (2.0 + z)
    u = w * w
    p = 1.0 + u * (1.0 / 3.0 + u * (0.2 + u * (1.0 / 7.0
        + u * (1.0 / 9.0 + u * (1.0 / 11.0)))))
    return jnp.minimum(x, 0.0) - 2.0 * w * p


def _pack16(vals, iota):
    acc = jnp.full((L,), vals[0], jnp.float32)
    for kk in range(1, L):
        acc = jnp.where(iota == kk, vals[kk], acc)
    return acc


def _row_l1(ref, row, hr):
    acc = jnp.abs(hr[0] - ref[row, pl.ds(0, L)])
    for c in range(1, D // L):
        acc = acc + jnp.abs(hr[c] - ref[row, pl.ds(c * L, L)])
    return jnp.sum(acc)


def _body(hidx_h, ridx_h, tidx_h, nidx_h, ent_h, rel_h,
          pos_h, neg_h,
          hidx_v, ridx_v, tidx_v, nidx_v,
          hrows, rrows, trows, negb,
          partials, posb, negbuf_out,
          sem_h, sem_r, sem_t, sem0, sem1, sem2, sem3, sem4):
    sems = (sem0, sem1, sem2, sem3, sem4)
    cid = lax.axis_index("c")
    sid = lax.axis_index("s")
    wid = sid * NC + cid
    base = wid * NB

    pltpu.sync_copy(hidx_h.at[pl.ds(base, NB)], hidx_v)
    pltpu.sync_copy(ridx_h.at[pl.ds(base, NB)], ridx_v)
    pltpu.sync_copy(tidx_h.at[pl.ds(base, NB)], tidx_v)
    pltpu.sync_copy(nidx_h.at[pl.ds(wid * NU, NU)], nidx_v)

    cp_h = pltpu.async_copy(ent_h.at[hidx_v], hrows, sem_h)
    cp_r = pltpu.async_copy(rel_h.at[ridx_v], rrows, sem_r)
    cp_t = pltpu.async_copy(ent_h.at[tidx_v], trows, sem_t)
    for u in range(NBUF - 1):
        pltpu.async_copy(ent_h.at[nidx_v.at[u]], negb.at[u], sems[u])

    cp_h.wait()
    cp_r.wait()
    iota = lax.iota(jnp.int32, L)

    def _hr_body(b, carry):
        for c in range(D // L):
            rrows[b, pl.ds(c * L, L)] = (
                rrows[b, pl.ds(c * L, L)] + hrows[b, pl.ds(c * L, L)])
        return carry
    lax.fori_loop(0, NB, _hr_body, 0)

    cp_t.wait()

    def _pos_body(g, carry):
        vals = []
        for kk in range(L):
            b = g * L + kk
            hr = [rrows[b, pl.ds(c * L, L)] for c in range(D // L)]
            vals.append(_row_l1(trows, b, hr))
        acc = _pack16(vals, iota)
        posb[pl.ds(g * L, L)] = _logsig(GAMMA - acc)
        return carry
    lax.fori_loop(0, NB // L, _pos_body, 0)

    def _unit(u, nb):
        pltpu.make_async_copy(ent_h.at[nidx_v.at[0]], negb.at[nb], sems[nb]).wait()
        buf = negb.at[nb]
        for j in range(GU):
            bl = u * GU + j
            hr = [rrows[bl, pl.ds(c * L, L)] for c in range(D // L)]

            def _gblk(g, lsacc):
                row0 = j * K + g * L
                vals = []
                for kk in range(L):
                    vals.append(_row_l1(buf, row0 + kk, hr))
                acc = _pack16(vals, iota)
                return lsacc + _logsig(acc - GAMMA)

            lsacc = lax.fori_loop(0, K // L, _gblk, jnp.zeros((L,), jnp.float32))
            partials[bl, :] = lsacc

        nxt = u + NBUF - 1
        slot = (nb + NBUF - 1) % NBUF

        if isinstance(nxt, int):
            if nxt < NU:
                pltpu.async_copy(ent_h.at[nidx_v.at[nxt]], negb.at[slot],
                                 sems[slot])
        else:
            @pl.when(nxt < NU)
            def _():
                pltpu.async_copy(ent_h.at[nidx_v.at[nxt]], negb.at[slot],
                                 sems[slot])

    def _ring_body(u0, carry):
        for nb in range(NBUF):
            _unit(u0 * NBUF + nb, nb)
        return carry
    lax.fori_loop(0, NU // NBUF, _ring_body, 0)

    for r in range(NU - (NU // NBUF) * NBUF):
        _unit((NU // NBUF) * NBUF + r, r)

    def _fin_body(g, carry):
        vals = []
        for kk in range(L):
            vals.append(jnp.sum(partials[g * L + kk, :]))
        negbuf_out[pl.ds(g * L, L)] = _pack16(vals, iota) * jnp.float32(1.0 / K)
        return carry
    lax.fori_loop(0, NB // L, _fin_body, 0)

    pltpu.sync_copy(posb, pos_h.at[pl.ds(base, NB)])
    pltpu.sync_copy(negbuf_out, neg_h.at[pl.ds(base, NB)])


_sc_score = functools.partial(
    pl.kernel,
    out_type=(jax.ShapeDtypeStruct((B,), jnp.float32),
              jax.ShapeDtypeStruct((B,), jnp.float32)),
    mesh=plsc.VectorSubcoreMesh(core_axis_name="c", subcore_axis_name="s",
                                num_cores=NC, num_subcores=NS),
    compiler_params=pltpu.CompilerParams(needs_layout_passes=False,
                                         use_tc_tiling_on_sc=False),
    scratch_types=[
        pltpu.VMEM((NB,), jnp.int32),
        pltpu.VMEM((NB,), jnp.int32),
        pltpu.VMEM((NB,), jnp.int32),
        pltpu.VMEM((NU, ROWS_PER_UNIT), jnp.int32),
        pltpu.VMEM((NB, D), jnp.float32),
        pltpu.VMEM((NB, D), jnp.float32),
        pltpu.VMEM((NB, D), jnp.float32),
        pltpu.VMEM((NBUF, ROWS_PER_UNIT, D), jnp.float32),
        pltpu.VMEM((NB, L), jnp.float32),
        pltpu.VMEM((NB,), jnp.float32),
        pltpu.VMEM((NB,), jnp.float32),
        pltpu.SemaphoreType.DMA,
        pltpu.SemaphoreType.DMA,
        pltpu.SemaphoreType.DMA,
        pltpu.SemaphoreType.DMA,
        pltpu.SemaphoreType.DMA,
        pltpu.SemaphoreType.DMA,
        pltpu.SemaphoreType.DMA,
        pltpu.SemaphoreType.DMA,
    ],
)(_body)


TR_LOG = 15
TR_C = 1 << TR_LOG
TR_H = TR_C // 2


def _tr_body(in_ref, out_ref):
    t = in_ref[...]
    out_ref[...] = jnp.concatenate([t[:, :TR_H], t[:, TR_H:]], axis=0).T


def _transpose_pack(entT):
    n = entT.shape[1]
    nblk = pl.cdiv(n, TR_C)
    return pl.pallas_call(
        _tr_body,
        grid=(nblk,),
        in_specs=[pl.BlockSpec((D, TR_C), lambda i: (0, i))],
        out_specs=pl.BlockSpec((TR_H, 128), lambda i: (i, 0)),
        out_shape=jax.ShapeDtypeStruct((nblk * TR_H, 128), jnp.float32),
        compiler_params=pltpu.CompilerParams(
            dimension_semantics=("parallel",)),
    )(entT)


def _remap(e):
    e = e.astype(jnp.int32)
    return (((e >> TR_LOG) << TR_LOG) + ((e & (TR_H - 1)) << 1)
            + ((e >> (TR_LOG - 1)) & 1))


def kernel(head_idx, rel_idx, tail_idx, neg_idx, entity_emb, relation_emb):
    ent128 = _transpose_pack(entity_emb.T)
    ent2 = ent128.reshape(2 * ent128.shape[0], D)
    neg_idx = _remap(neg_idx).reshape(B * K // ROWS_PER_UNIT, ROWS_PER_UNIT)
    pos, neg = _sc_score(_remap(head_idx), rel_idx.astype(jnp.int32),
                         _remap(tail_idx), neg_idx, ent2, relation_emb)
    return jnp.stack([pos, neg], axis=1)


# speedup vs baseline: 1.0580x; 1.0300x over previous
import functools

import jax
import jax.numpy as jnp
from jax import lax
from jax.experimental import pallas as pl
from jax.experimental.pallas import tpu as pltpu
from jax.experimental.pallas import tpu_sc as plsc

B = 4096
K = 64
D = 64
GAMMA = 12.0

NC = 2
NS = 16
L = 16
NW = NC * NS
NB = B // NW
GU = 2
ROWS_PER_UNIT = GU * K
NU = NB // GU
NBUF = 4


def _logsig(x):
    ax = jnp.abs(x)
    z = jnp.exp(-ax)
    w = z / (2.0 + z)
    u = w * w
    p = 1.0 + u * (1.0 / 3.0 + u * (0.2 + u * (1.0 / 7.0
        + u * (1.0 / 9.0 + u * (1.0 / 11.0)))))
    return jnp.minimum(x, 0.0) - 2.0 * w * p


def _pack16(vals, iota):
    acc = jnp.full((L,), vals[0], jnp.float32)
    for kk in range(1, L):
        acc = jnp.where(iota == kk, vals[kk], acc)
    return acc


def _row_l1(ref, row, hr):
    acc = jnp.abs(hr[0] - ref[row, pl.ds(0, L)])
    for c in range(1, D // L):
        acc = acc + jnp.abs(hr[c] - ref[row, pl.ds(c * L, L)])
    return jnp.sum(acc)


def _body(hidx_h, ridx_h, tidx_h, nidx_h, ent_h, rel_h,
          pos_h, neg_h,
          hidx_v, ridx_v, tidx_v, nidx_v,
          hrows, rrows, trows, negb,
          partials, posb, negbuf_out,
          sem_h, sem_r, sem_t, sem0, sem1, sem2, sem3):
    sems = (sem0, sem1, sem2, sem3)
    cid = lax.axis_index("c")
    sid = lax.axis_index("s")
    wid = sid * NC + cid
    base = wid * NB

    pltpu.sync_copy(hidx_h.at[pl.ds(base, NB)], hidx_v)
    pltpu.sync_copy(ridx_h.at[pl.ds(base, NB)], ridx_v)
    pltpu.sync_copy(tidx_h.at[pl.ds(base, NB)], tidx_v)
    pltpu.sync_copy(nidx_h.at[pl.ds(wid * NU, NU)], nidx_v)

    cp_h = pltpu.async_copy(ent_h.at[hidx_v], hrows, sem_h)
    cp_r = pltpu.async_copy(rel_h.at[ridx_v], rrows, sem_r)
    cp_t = pltpu.async_copy(ent_h.at[tidx_v], trows, sem_t)
    for u in range(NBUF - 1):
        pltpu.async_copy(ent_h.at[nidx_v.at[u]], negb.at[u], sems[u])

    cp_h.wait()
    cp_r.wait()
    iota = lax.iota(jnp.int32, L)

    def _hr_body(b, carry):
        for c in range(D // L):
            rrows[b, pl.ds(c * L, L)] = (
                rrows[b, pl.ds(c * L, L)] + hrows[b, pl.ds(c * L, L)])
        return carry
    lax.fori_loop(0, NB, _hr_body, 0)

    cp_t.wait()

    def _pos_body(g, carry):
        vals = []
        for kk in range(L):
            b = g * L + kk
            hr = [rrows[b, pl.ds(c * L, L)] for c in range(D // L)]
            vals.append(_row_l1(trows, b, hr))
        acc = _pack16(vals, iota)
        posb[pl.ds(g * L, L)] = _logsig(GAMMA - acc)
        return carry
    lax.fori_loop(0, NB // L, _pos_body, 0)

    def _unit(u, nb):
        pltpu.make_async_copy(ent_h.at[nidx_v.at[0]], negb.at[nb], sems[nb]).wait()
        buf = negb.at[nb]
        for j in range(GU):
            bl = u * GU + j
            hr = [rrows[bl, pl.ds(c * L, L)] for c in range(D // L)]

            def _gblk(g, lsacc):
                row0 = j * K + g * L
                vals = []
                for kk in range(L):
                    vals.append(_row_l1(buf, row0 + kk, hr))
                acc = _pack16(vals, iota)
                return lsacc + _logsig(acc - GAMMA)

            lsacc = lax.fori_loop(0, K // L, _gblk, jnp.zeros((L,), jnp.float32))
            partials[bl, :] = lsacc

        nxt = u + NBUF - 1
        slot = (nb + NBUF - 1) % NBUF

        if isinstance(nxt, int):
            if nxt < NU:
                pltpu.async_copy(ent_h.at[nidx_v.at[nxt]], negb.at[slot],
                                 sems[slot])
        else:
            @pl.when(nxt < NU)
            def _():
                pltpu.async_copy(ent_h.at[nidx_v.at[nxt]], negb.at[slot],
                                 sems[slot])

    def _ring_body(u0, carry):
        for nb in range(NBUF):
            _unit(u0 * NBUF + nb, nb)
        return carry
    lax.fori_loop(0, NU // NBUF, _ring_body, 0)

    for r in range(NU - (NU // NBUF) * NBUF):
        _unit((NU // NBUF) * NBUF + r, r)

    def _fin_body(g, carry):
        vals = []
        for kk in range(L):
            vals.append(jnp.sum(partials[g * L + kk, :]))
        negbuf_out[pl.ds(g * L, L)] = _pack16(vals, iota) * jnp.float32(1.0 / K)
        return carry
    lax.fori_loop(0, NB // L, _fin_body, 0)

    pltpu.sync_copy(posb, pos_h.at[pl.ds(base, NB)])
    pltpu.sync_copy(negbuf_out, neg_h.at[pl.ds(base, NB)])


_sc_score = functools.partial(
    pl.kernel,
    out_type=(jax.ShapeDtypeStruct((B,), jnp.float32),
              jax.ShapeDtypeStruct((B,), jnp.float32)),
    mesh=plsc.VectorSubcoreMesh(core_axis_name="c", subcore_axis_name="s",
                                num_cores=NC, num_subcores=NS),
    compiler_params=pltpu.CompilerParams(needs_layout_passes=False,
                                         use_tc_tiling_on_sc=False),
    scratch_types=[
        pltpu.VMEM((NB,), jnp.int32),
        pltpu.VMEM((NB,), jnp.int32),
        pltpu.VMEM((NB,), jnp.int32),
        pltpu.VMEM((NU, ROWS_PER_UNIT), jnp.int32),
        pltpu.VMEM((NB, D), jnp.float32),
        pltpu.VMEM((NB, D), jnp.float32),
        pltpu.VMEM((NB, D), jnp.float32),
        pltpu.VMEM((NBUF, ROWS_PER_UNIT, D), jnp.float32),
        pltpu.VMEM((NB, L), jnp.float32),
        pltpu.VMEM((NB,), jnp.float32),
        pltpu.VMEM((NB,), jnp.float32),
        pltpu.SemaphoreType.DMA,
        pltpu.SemaphoreType.DMA,
        pltpu.SemaphoreType.DMA,
        pltpu.SemaphoreType.DMA,
        pltpu.SemaphoreType.DMA,
        pltpu.SemaphoreType.DMA,
        pltpu.SemaphoreType.DMA,
    ],
)(_body)


TR_LOG = 15
TR_C = 1 << TR_LOG
TR_H = TR_C // 2


def _tr_body(in_ref, out_ref):
    t = in_ref[...]
    out_ref[...] = jnp.concatenate([t[:, :TR_H], t[:, TR_H:]], axis=0).T


def _transpose_pack(entT):
    n = entT.shape[1]
    nblk = pl.cdiv(n, TR_C)
    return pl.pallas_call(
        _tr_body,
        grid=(nblk,),
        in_specs=[pl.BlockSpec((D, TR_C), lambda i: (0, i))],
        out_specs=pl.BlockSpec((TR_H, 128), lambda i: (i, 0)),
        out_shape=jax.ShapeDtypeStruct((nblk * TR_H, 128), jnp.float32),
        compiler_params=pltpu.CompilerParams(
            dimension_semantics=("parallel",)),
    )(entT)


def _remap(e):
    e = e.astype(jnp.int32)
    return (((e >> TR_LOG) << TR_LOG) + ((e & (TR_H - 1)) << 1)
            + ((e >> (TR_LOG - 1)) & 1))


def kernel(head_idx, rel_idx, tail_idx, neg_idx, entity_emb, relation_emb):
    ent128 = _transpose_pack(entity_emb.T)
    ent2 = ent128.reshape(2 * ent128.shape[0], D)
    neg_idx = _remap(neg_idx).reshape(B * K // ROWS_PER_UNIT, ROWS_PER_UNIT)
    pos, neg = _sc_score(_remap(head_idx), rel_idx.astype(jnp.int32),
                         _remap(tail_idx), neg_idx, ent2, relation_emb)
    return jnp.stack([pos, neg], axis=1)
